# Initial kernel scaffold; baseline (speedup 1.0000x reference)
import functools

import jax
import jax.numpy as jnp
from jax import lax
from jax.experimental import pallas as pl
from jax.experimental.pallas import tpu as pltpu
from jax.experimental.pallas import tpu_sc as plsc

N = 10000
E = 160000
FEAT = 256
EMBED = 256
HALF = 128

NS = 16
CH = 128
E_PAD = ((E + NS * CH - 1) // (NS * CH)) * (NS * CH)
EPT = E_PAD // NS
NCHUNK = EPT // CH
NACC = 10240
ZPT = NACC // NS
OPT = N // NS


def _gemm_body(x_ref, w_ref, o0_ref, o1_ref):
    h = lax.dot_general(x_ref[...], w_ref[...], (((1,), (1,)), ((), ())),
                        preferred_element_type=jnp.float32)
    o0_ref[...] = h[:, :HALF]
    o1_ref[...] = h[:, HALF:]


def _gemm(x, w):
    br = 1000
    return pl.pallas_call(
        _gemm_body,
        grid=(N // br,),
        in_specs=[
            pl.BlockSpec((br, FEAT), lambda i: (i, 0)),
            pl.BlockSpec((EMBED, FEAT), lambda i: (0, 0)),
        ],
        out_specs=[
            pl.BlockSpec((br, HALF), lambda i: (i, 0)),
            pl.BlockSpec((br, HALF), lambda i: (i, 0)),
        ],
        out_shape=[
            jax.ShapeDtypeStruct((N, HALF), jnp.float32),
            jax.ShapeDtypeStruct((N, HALF), jnp.float32),
        ],
    )(x, w)


_MESH = plsc.VectorSubcoreMesh(core_axis_name="c", subcore_axis_name="s")


@functools.partial(
    pl.kernel,
    mesh=_MESH,
    out_type=(
        jax.ShapeDtypeStruct((N, HALF), jnp.float32),
        jax.ShapeDtypeStruct((N, HALF), jnp.float32),
    ),
    scratch_types=[
        pltpu.VMEM((CH,), jnp.int32),
        pltpu.VMEM((CH,), jnp.int32),
        pltpu.VMEM((CH, HALF), jnp.float32),
        pltpu.VMEM_SHARED((NACC, HALF), jnp.float32),
        pltpu.SemaphoreType.DMA,
    ],
)
def _sc_aggregate(h0, h1, srcp, dstp, zrows, out0, out1,
                  src_v, dst_v, rows_v, acc, sem):
    c = lax.axis_index("c")
    s = lax.axis_index("s")

    pltpu.sync_copy(zrows, acc.at[pl.ds(s * ZPT, ZPT)])
    plsc.subcore_barrier()

    def run(h_ref, out_ref):
        def body(i, carry):
            base = s * EPT + i * CH
            pltpu.sync_copy(srcp.at[pl.ds(base, CH)], src_v)
            pltpu.sync_copy(dstp.at[pl.ds(base, CH)], dst_v)
            pltpu.async_copy(h_ref.at[src_v], rows_v, sem).wait()
            pltpu.sync_copy(rows_v, acc.at[dst_v], add=True)
            return carry

        lax.fori_loop(0, NCHUNK, body, 0)
        plsc.subcore_barrier()
        r0 = s * OPT
        pltpu.sync_copy(acc.at[pl.ds(r0, OPT)], out_ref.at[pl.ds(r0, OPT)])

    @pl.when(c == 0)
    def _():
        run(h0, out0)

    @pl.when(c == 1)
    def _():
        run(h1, out1)


def kernel(x, edge_index, W):
    h0, h1 = _gemm(x, W)
    pad = E_PAD - E
    srcp = jnp.concatenate([edge_index[0], jnp.zeros((pad,), jnp.int32)])
    dstp = jnp.concatenate([edge_index[1], jnp.full((pad,), N, jnp.int32)])
    zrows = jnp.zeros((ZPT, HALF), jnp.float32)
    o0, o1 = _sc_aggregate(h0, h1, srcp, dstp, zrows)
    return jnp.concatenate([o0, o1], axis=1)


# speedup vs baseline: 3.5880x; 3.5880x over previous
import functools

import jax
import jax.numpy as jnp
from jax import lax
from jax.experimental import pallas as pl
from jax.experimental.pallas import tpu as pltpu
from jax.experimental.pallas import tpu_sc as plsc

N = 10000
E = 160000
FEAT = 256
EMBED = 256
HALF = 128

NS = 16
CH = 128
E_PAD = ((E + NS * CH - 1) // (NS * CH)) * (NS * CH)
EPT = E_PAD // NS
NCHUNK = EPT // CH
NACC = 10240
ZPT = NACC // NS
OPT = 632
OPT_LAST = N - 15 * OPT


def _gemm_body(x_ref, w_ref, o0_ref, o1_ref):
    h = lax.dot_general(x_ref[...], w_ref[...], (((1,), (1,)), ((), ())),
                        preferred_element_type=jnp.float32)
    o0_ref[...] = h[:, :HALF]
    o1_ref[...] = h[:, HALF:]


def _gemm(x, w):
    br = 1000
    return pl.pallas_call(
        _gemm_body,
        grid=(N // br,),
        in_specs=[
            pl.BlockSpec((br, FEAT), lambda i: (i, 0)),
            pl.BlockSpec((EMBED, FEAT), lambda i: (0, 0)),
        ],
        out_specs=[
            pl.BlockSpec((br, HALF), lambda i: (i, 0)),
            pl.BlockSpec((br, HALF), lambda i: (i, 0)),
        ],
        out_shape=[
            jax.ShapeDtypeStruct((N, HALF), jnp.float32),
            jax.ShapeDtypeStruct((N, HALF), jnp.float32),
        ],
    )(x, w)


_MESH = plsc.VectorSubcoreMesh(core_axis_name="c", subcore_axis_name="s")


@functools.partial(
    pl.kernel,
    mesh=_MESH,
    out_type=(
        jax.ShapeDtypeStruct((N, HALF), jnp.float32),
        jax.ShapeDtypeStruct((N, HALF), jnp.float32),
    ),
    scratch_types=[
        pltpu.VMEM((CH,), jnp.int32),
        pltpu.VMEM((CH,), jnp.int32),
        pltpu.VMEM((CH, HALF), jnp.float32),
        pltpu.VMEM_SHARED((NACC, HALF), jnp.float32),
        pltpu.SemaphoreType.DMA,
    ],
)
def _sc_aggregate(h0, h1, srcp, dstp, zrows, out0, out1,
                  src_v, dst_v, rows_v, acc, sem):
    c = lax.axis_index("c")
    s = lax.axis_index("s")

    pltpu.sync_copy(zrows, acc.at[pl.ds(s * ZPT, ZPT)])
    plsc.subcore_barrier()

    def run(h_ref, out_ref):
        def body(i, carry):
            base = s * EPT + i * CH
            pltpu.sync_copy(srcp.at[pl.ds(base, CH)], src_v)
            pltpu.sync_copy(dstp.at[pl.ds(base, CH)], dst_v)
            pltpu.async_copy(h_ref.at[src_v], rows_v, sem).wait()
            pltpu.sync_copy(rows_v, acc.at[dst_v], add=True)
            return carry

        lax.fori_loop(0, NCHUNK, body, 0)
        plsc.subcore_barrier()
        r0 = s * OPT

        @pl.when(s < NS - 1)
        def _():
            pltpu.sync_copy(acc.at[pl.ds(r0, OPT)], out_ref.at[pl.ds(r0, OPT)])

        @pl.when(s == NS - 1)
        def _():
            pltpu.sync_copy(acc.at[pl.ds(15 * OPT, OPT_LAST)],
                            out_ref.at[pl.ds(15 * OPT, OPT_LAST)])

    @pl.when(c == 0)
    def _():
        run(h0, out0)

    @pl.when(c == 1)
    def _():
        run(h1, out1)


def kernel(x, edge_index, W):
    h0, h1 = _gemm(x, W)
    pad = E_PAD - E
    srcp = jnp.concatenate([edge_index[0], jnp.zeros((pad,), jnp.int32)])
    dstp = jnp.concatenate([edge_index[1], jnp.full((pad,), N, jnp.int32)])
    zrows = jnp.zeros((ZPT, HALF), jnp.float32)
    o0, o1 = _sc_aggregate(h0, h1, srcp, dstp, zrows)
    return jnp.concatenate([o0, o1], axis=1)
